# Initial kernel scaffold; baseline (speedup 1.0000x reference)
import jax
import jax.numpy as jnp
from jax.experimental import pallas as pl


def kernel(x, params, edge_index, batch):
    raise NotImplementedError("write your pallas kernel here")


# speedup vs baseline: 5.0385x; 5.0385x over previous
import functools

import numpy as np
import jax
import jax.numpy as jnp
from jax import lax
from jax.experimental import pallas as pl
from jax.experimental.pallas import tpu as pltpu
from jax.experimental.pallas import tpu_sc as plsc

N = 10000
E = 320000
G = 64
H = 128
C = 10
EPS = 1e-5

NC = 2
NS = 16
NW = NC * NS
HS = H // NC
CH = 128
EPT = E // NW
NCHUNKD = 80
EPTP = NCHUNKD * CH
NCHUNK = 2 * NCHUNKD
NBUF = 4
NGROUP = NCHUNK // NBUF
NP = 10240
RPW = NP // NS

R = 1000
NBLK = N // R


@functools.cache
def _make_agg():
    mesh = plsc.VectorSubcoreMesh(core_axis_name="c", subcore_axis_name="s")

    def body(xs_hbm, src_hbm, dst_hbm, zrow_hbm,
             acc_out, src_v, dst_v, rows_v, acc_sh, sem0, sem1, sem2, sem3):
        sems = [sem0, sem1, sem2, sem3]
        cid = lax.axis_index("c")
        sid = lax.axis_index("s")
        pltpu.sync_copy(src_hbm.at[sid], src_v)
        pltpu.sync_copy(dst_hbm.at[sid], dst_v)
        row0 = sid * RPW
        pltpu.sync_copy(zrow_hbm, acc_sh.at[pl.ds(row0, RPW)])
        plsc.subcore_barrier()
        table = xs_hbm.at[cid]

        def group(g, carry):
            handles = []
            for b in range(NBUF):
                ch = g * NBUF + b
                handles.append(
                    pltpu.async_copy(table.at[src_v.at[ch]], rows_v.at[b], sems[b]))
            for b in range(NBUF):
                ch = g * NBUF + b
                handles[b].wait()
                pltpu.sync_copy(rows_v.at[b], acc_sh.at[dst_v.at[ch]], add=True)
            return carry

        lax.fori_loop(0, NGROUP, group, 0)
        plsc.subcore_barrier()
        pltpu.sync_copy(acc_sh.at[pl.ds(row0, RPW)],
                        acc_out.at[cid].at[pl.ds(row0, RPW)])

    return pl.kernel(
        body, mesh=mesh,
        out_type=[jax.ShapeDtypeStruct((NC, NP, HS), jnp.float32)],
        compiler_params=pltpu.CompilerParams(use_tc_tiling_on_sc=False),
        scratch_types=[
            pltpu.VMEM((NCHUNK, CH), jnp.int32),
            pltpu.VMEM((NCHUNK, CH), jnp.int32),
            pltpu.VMEM((NBUF, CH, HS), jnp.float32),
            pltpu.VMEM_SHARED((NP, HS), jnp.float32),
        ] + [pltpu.SemaphoreType.DMA] * NBUF)


@functools.cache
def _make_deg():
    mesh = plsc.VectorSubcoreMesh(core_axis_name="c", subcore_axis_name="s")

    def body(dst_hbm, zdeg_hbm, ones_hbm,
             deg_out, dst_v, ones_v, deg_sh):
        cid = lax.axis_index("c")
        sid = lax.axis_index("s")
        tg = cid * NS + sid
        pltpu.sync_copy(dst_hbm.at[tg], dst_v)
        row0 = sid * RPW
        pltpu.sync_copy(zdeg_hbm, deg_sh.at[pl.ds(row0, RPW)])
        pltpu.sync_copy(ones_hbm, ones_v)
        plsc.subcore_barrier()

        def chunk(j, carry):
            pltpu.sync_copy(ones_v, deg_sh.at[dst_v.at[j]], add=True)
            return carry

        lax.fori_loop(0, NCHUNKD, chunk, 0)
        plsc.subcore_barrier()
        pltpu.sync_copy(deg_sh.at[pl.ds(row0, RPW)],
                        deg_out.at[cid].at[pl.ds(row0, RPW)])

    return pl.kernel(
        body, mesh=mesh,
        out_type=[jax.ShapeDtypeStruct((NC, NP, 8), jnp.float32)],
        compiler_params=pltpu.CompilerParams(use_tc_tiling_on_sc=False),
        scratch_types=[
            pltpu.VMEM((NCHUNKD, CH), jnp.int32),
            pltpu.VMEM((CH, 8), jnp.float32),
            pltpu.VMEM_SHARED((NP, 8), jnp.float32),
        ])


def _dense_body(acc_ref, deg_ref, x_ref, wn_ref, wr_ref, c_ref, out_ref):
    d = deg_ref[0, :, 0:1] + deg_ref[1, :, 0:1]
    s = jnp.concatenate([acc_ref[0, :, :], acc_ref[1, :, :]], axis=1)
    xc = jnp.concatenate([x_ref[0, :, :], x_ref[1, :, :]], axis=1)
    mean = s / jnp.maximum(d, 1.0)
    y = (jnp.dot(mean, wn_ref[...], preferred_element_type=jnp.float32)
         + jnp.dot(xc, wr_ref[...], preferred_element_type=jnp.float32)
         + c_ref[...])
    y = jnp.maximum(y, 0.0)
    out_ref[0, :, :] = y[:, :HS]
    out_ref[1, :, :] = y[:, HS:]


def _dense(acc, deg, x, wn, wr, c):
    return pl.pallas_call(
        _dense_body,
        grid=(NBLK,),
        in_specs=[
            pl.BlockSpec((NC, R, HS), lambda i: (0, i, 0)),
            pl.BlockSpec((NC, R, 8), lambda i: (0, i, 0)),
            pl.BlockSpec((NC, R, HS), lambda i: (0, i, 0)),
            pl.BlockSpec((H, H), lambda i: (0, 0)),
            pl.BlockSpec((H, H), lambda i: (0, 0)),
            pl.BlockSpec((1, H), lambda i: (0, 0)),
        ],
        out_specs=pl.BlockSpec((NC, R, HS), lambda i: (0, i, 0)),
        out_shape=jax.ShapeDtypeStruct((NC, N, HS), jnp.float32),
    )(acc, deg, x, wn, wr, c)


def _final_body(x_ref, batch_ref,
                w1_ref, b1_ref, w2_ref, b2_ref, out_ref, pooled_s, cnt_s):
    i = pl.program_id(0)
    feats = jnp.concatenate([x_ref[0, :, :], x_ref[1, :, :]], axis=1)
    b = batch_ref[...]
    gid = lax.broadcasted_iota(jnp.int32, (1, G), 1)
    onehot = (b == gid).astype(jnp.float32)

    @pl.when(i == 0)
    def _():
        pooled_s[...] = jnp.zeros_like(pooled_s)
        cnt_s[...] = jnp.zeros_like(cnt_s)

    dn = (((0,), (0,)), ((), ()))
    pooled_s[...] += lax.dot_general(onehot, feats, dn,
                                     preferred_element_type=jnp.float32)
    cnt_s[...] += lax.dot_general(onehot, jnp.ones((R, H), jnp.float32), dn,
                                  preferred_element_type=jnp.float32)

    @pl.when(i == NBLK - 1)
    def _():
        pooled = pooled_s[...] / jnp.maximum(cnt_s[...], 1.0)
        h = jnp.maximum(
            jnp.dot(pooled, w1_ref[...], preferred_element_type=jnp.float32)
            + b1_ref[...], 0.0)
        out_ref[...] = (jnp.dot(h, w2_ref[...],
                                preferred_element_type=jnp.float32)
                        + b2_ref[...])


def _final(x, batch2, w1, b1, w2, b2):
    return pl.pallas_call(
        _final_body,
        grid=(NBLK,),
        in_specs=[
            pl.BlockSpec((NC, R, HS), lambda i: (0, i, 0)),
            pl.BlockSpec((R, 1), lambda i: (i, 0)),
            pl.BlockSpec((H, H), lambda i: (0, 0)),
            pl.BlockSpec((1, H), lambda i: (0, 0)),
            pl.BlockSpec((H, C), lambda i: (0, 0)),
            pl.BlockSpec((1, C), lambda i: (0, 0)),
        ],
        out_specs=pl.BlockSpec((G, C), lambda i: (0, 0)),
        out_shape=jax.ShapeDtypeStruct((G, C), jnp.float32),
        scratch_shapes=[
            pltpu.VMEM((G, H), jnp.float32),
            pltpu.VMEM((G, H), jnp.float32),
        ],
    )(x, batch2, w1, b1, w2, b2)


def kernel(x, params, edge_index, batch):
    pad = EPTP - EPT
    srcp = jnp.pad(edge_index[0].reshape(NW, EPT), ((0, 0), (0, pad)))
    dstp = jnp.pad(edge_index[1].reshape(NW, EPT),
                   ((0, 0), (0, pad)), constant_values=N)
    src3 = srcp.reshape(NS, NCHUNK, CH)
    dst3 = dstp.reshape(NS, NCHUNK, CH)
    dst3d = dstp.reshape(NW, NCHUNKD, CH)
    batch2 = batch.reshape(N, 1)
    xs = x.reshape(N, NC, HS).transpose(1, 0, 2)
    zrow = jnp.zeros((RPW, HS), jnp.float32)
    zdeg = jnp.zeros((RPW, 8), jnp.float32)
    ones8 = jnp.ones((CH, 8), jnp.float32)
    gscale = np.float32(1.0 / np.sqrt(1.0 + EPS))

    wns, wrs, cs = [], [], []
    for lp in params["layers"]:
        g = lp["gamma"] * gscale
        wns.append(lp["Wn"] * g[None, :])
        wrs.append(lp["Wr"] * g[None, :])
        cs.append((lp["bn"] * g + lp["beta"])[None, :])
    wns = jnp.stack(wns)
    wrs = jnp.stack(wrs)
    cs = jnp.stack(cs)

    deg = _make_deg()(dst3d, zdeg, ones8)
    if isinstance(deg, (list, tuple)):
        deg = deg[0]

    def step(feats, ws):
        wn2, wr2, c2 = ws
        acc = _make_agg()(feats, src3, dst3, zrow)
        if isinstance(acc, (list, tuple)):
            acc = acc[0]
        return _dense(acc, deg, feats, wn2, wr2, c2), None

    feats, _ = lax.scan(step, xs, (wns, wrs, cs))
    hd = params["head"]
    return _final(feats, batch2,
                  hd["W1"], hd["b1"][None], hd["W2"], hd["b2"][None])


# speedup vs baseline: 5.8673x; 1.1645x over previous
import functools

import numpy as np
import jax
import jax.numpy as jnp
from jax import lax
from jax.experimental import pallas as pl
from jax.experimental.pallas import tpu as pltpu
from jax.experimental.pallas import tpu_sc as plsc

N = 10000
E = 320000
G = 64
H = 128
C = 10
EPS = 1e-5

NC = 2
NS = 16
NW = NC * NS
HS = H // NC
CH = 128
EPT = E // NW
NCHUNKD = 80
EPTP = NCHUNKD * CH
NCHUNK = 2 * NCHUNKD
NBUF = 5
NGROUP = NCHUNK // NBUF
NP = 10240
RPW = NP // NS

R = 1000
NBLK = N // R


@functools.cache
def _make_agg():
    mesh = plsc.VectorSubcoreMesh(core_axis_name="c", subcore_axis_name="s")

    def body(xs_hbm, src_hbm, dst_hbm, zrow_hbm,
             acc_out, src_v, dst_v, rows_v, acc_sh, *sems):
        gsems = sems[:NBUF]
        ssems = sems[NBUF:]
        cid = lax.axis_index("c")
        sid = lax.axis_index("s")
        pltpu.sync_copy(src_hbm.at[sid], src_v)
        pltpu.sync_copy(dst_hbm.at[sid], dst_v)
        row0 = sid * RPW
        pltpu.sync_copy(zrow_hbm, acc_sh.at[pl.ds(row0, RPW)])
        plsc.subcore_barrier()
        table = xs_hbm.at[cid]

        def gather(ch, b):
            pltpu.async_copy(table.at[src_v.at[ch]], rows_v.at[b], gsems[b])

        for b in range(NBUF):
            gather(b, b)

        def group(g, carry):
            for b in range(NBUF):
                ch = g * NBUF + b
                pltpu.make_async_copy(
                    table.at[src_v.at[ch]], rows_v.at[b], gsems[b]).wait()
                pltpu.async_copy(
                    rows_v.at[b], acc_sh.at[dst_v.at[ch]], ssems[b], add=True)
                nxt = ch + NBUF

                @pl.when(nxt < NCHUNK)
                def _():
                    pltpu.make_async_copy(
                        rows_v.at[b], acc_sh.at[dst_v.at[ch]], ssems[b]).wait()
                    gather(nxt, b)
            return carry

        lax.fori_loop(0, NGROUP, group, 0)
        for b in range(NBUF):
            pltpu.make_async_copy(
                rows_v.at[b], acc_sh.at[dst_v.at[0]], ssems[b]).wait()
        plsc.subcore_barrier()
        pltpu.sync_copy(acc_sh.at[pl.ds(row0, RPW)],
                        acc_out.at[cid].at[pl.ds(row0, RPW)])

    return pl.kernel(
        body, mesh=mesh,
        out_type=[jax.ShapeDtypeStruct((NC, NP, HS), jnp.float32)],
        compiler_params=pltpu.CompilerParams(use_tc_tiling_on_sc=False),
        scratch_types=[
            pltpu.VMEM((NCHUNK, CH), jnp.int32),
            pltpu.VMEM((NCHUNK, CH), jnp.int32),
            pltpu.VMEM((NBUF, CH, HS), jnp.float32),
            pltpu.VMEM_SHARED((NP, HS), jnp.float32),
        ] + [pltpu.SemaphoreType.DMA] * (2 * NBUF))


@functools.cache
def _make_deg():
    mesh = plsc.VectorSubcoreMesh(core_axis_name="c", subcore_axis_name="s")

    def body(dst_hbm, zdeg_hbm, ones_hbm,
             deg_out, dst_v, ones_v, deg_sh):
        cid = lax.axis_index("c")
        sid = lax.axis_index("s")
        tg = cid * NS + sid
        pltpu.sync_copy(dst_hbm.at[tg], dst_v)
        row0 = sid * RPW
        pltpu.sync_copy(zdeg_hbm, deg_sh.at[pl.ds(row0, RPW)])
        pltpu.sync_copy(ones_hbm, ones_v)
        plsc.subcore_barrier()

        def chunk(j, carry):
            pltpu.sync_copy(ones_v, deg_sh.at[dst_v.at[j]], add=True)
            return carry

        lax.fori_loop(0, NCHUNKD, chunk, 0)
        plsc.subcore_barrier()
        pltpu.sync_copy(deg_sh.at[pl.ds(row0, RPW)],
                        deg_out.at[cid].at[pl.ds(row0, RPW)])

    return pl.kernel(
        body, mesh=mesh,
        out_type=[jax.ShapeDtypeStruct((NC, NP, 8), jnp.float32)],
        compiler_params=pltpu.CompilerParams(use_tc_tiling_on_sc=False),
        scratch_types=[
            pltpu.VMEM((NCHUNKD, CH), jnp.int32),
            pltpu.VMEM((CH, 8), jnp.float32),
            pltpu.VMEM_SHARED((NP, 8), jnp.float32),
        ])


def _dense_body(acc_ref, deg_ref, x_ref, wn_ref, wr_ref, c_ref, out_ref):
    d = deg_ref[0, :, 0:1] + deg_ref[1, :, 0:1]
    s = jnp.concatenate([acc_ref[0, :, :], acc_ref[1, :, :]], axis=1)
    xc = jnp.concatenate([x_ref[0, :, :], x_ref[1, :, :]], axis=1)
    mean = s / jnp.maximum(d, 1.0)
    y = (jnp.dot(mean, wn_ref[...], preferred_element_type=jnp.float32)
         + jnp.dot(xc, wr_ref[...], preferred_element_type=jnp.float32)
         + c_ref[...])
    y = jnp.maximum(y, 0.0)
    out_ref[0, :, :] = y[:, :HS]
    out_ref[1, :, :] = y[:, HS:]


def _dense(acc, deg, x, wn, wr, c):
    return pl.pallas_call(
        _dense_body,
        grid=(NBLK,),
        in_specs=[
            pl.BlockSpec((NC, R, HS), lambda i: (0, i, 0)),
            pl.BlockSpec((NC, R, 8), lambda i: (0, i, 0)),
            pl.BlockSpec((NC, R, HS), lambda i: (0, i, 0)),
            pl.BlockSpec((H, H), lambda i: (0, 0)),
            pl.BlockSpec((H, H), lambda i: (0, 0)),
            pl.BlockSpec((1, H), lambda i: (0, 0)),
        ],
        out_specs=pl.BlockSpec((NC, R, HS), lambda i: (0, i, 0)),
        out_shape=jax.ShapeDtypeStruct((NC, N, HS), jnp.float32),
    )(acc, deg, x, wn, wr, c)


def _final_body(x_ref, batch_ref,
                w1_ref, b1_ref, w2_ref, b2_ref, out_ref, pooled_s, cnt_s):
    i = pl.program_id(0)
    feats = jnp.concatenate([x_ref[0, :, :], x_ref[1, :, :]], axis=1)
    b = batch_ref[...]
    gid = lax.broadcasted_iota(jnp.int32, (1, G), 1)
    onehot = (b == gid).astype(jnp.float32)

    @pl.when(i == 0)
    def _():
        pooled_s[...] = jnp.zeros_like(pooled_s)
        cnt_s[...] = jnp.zeros_like(cnt_s)

    dn = (((0,), (0,)), ((), ()))
    pooled_s[...] += lax.dot_general(onehot, feats, dn,
                                     preferred_element_type=jnp.float32)
    cnt_s[...] += lax.dot_general(onehot, jnp.ones((R, H), jnp.float32), dn,
                                  preferred_element_type=jnp.float32)

    @pl.when(i == NBLK - 1)
    def _():
        pooled = pooled_s[...] / jnp.maximum(cnt_s[...], 1.0)
        h = jnp.maximum(
            jnp.dot(pooled, w1_ref[...], preferred_element_type=jnp.float32)
            + b1_ref[...], 0.0)
        out_ref[...] = (jnp.dot(h, w2_ref[...],
                                preferred_element_type=jnp.float32)
                        + b2_ref[...])


def _final(x, batch2, w1, b1, w2, b2):
    return pl.pallas_call(
        _final_body,
        grid=(NBLK,),
        in_specs=[
            pl.BlockSpec((NC, R, HS), lambda i: (0, i, 0)),
            pl.BlockSpec((R, 1), lambda i: (i, 0)),
            pl.BlockSpec((H, H), lambda i: (0, 0)),
            pl.BlockSpec((1, H), lambda i: (0, 0)),
            pl.BlockSpec((H, C), lambda i: (0, 0)),
            pl.BlockSpec((1, C), lambda i: (0, 0)),
        ],
        out_specs=pl.BlockSpec((G, C), lambda i: (0, 0)),
        out_shape=jax.ShapeDtypeStruct((G, C), jnp.float32),
        scratch_shapes=[
            pltpu.VMEM((G, H), jnp.float32),
            pltpu.VMEM((G, H), jnp.float32),
        ],
    )(x, batch2, w1, b1, w2, b2)


def kernel(x, params, edge_index, batch):
    pad = EPTP - EPT
    srcp = jnp.pad(edge_index[0].reshape(NW, EPT), ((0, 0), (0, pad)))
    dstp = jnp.pad(edge_index[1].reshape(NW, EPT),
                   ((0, 0), (0, pad)), constant_values=N)
    src3 = srcp.reshape(NS, NCHUNK, CH)
    dst3 = dstp.reshape(NS, NCHUNK, CH)
    dst3d = dstp.reshape(NW, NCHUNKD, CH)
    batch2 = batch.reshape(N, 1)
    xs = x.reshape(N, NC, HS).transpose(1, 0, 2)
    zrow = jnp.zeros((RPW, HS), jnp.float32)
    zdeg = jnp.zeros((RPW, 8), jnp.float32)
    ones8 = jnp.ones((CH, 8), jnp.float32)
    gscale = np.float32(1.0 / np.sqrt(1.0 + EPS))

    wns, wrs, cs = [], [], []
    for lp in params["layers"]:
        g = lp["gamma"] * gscale
        wns.append(lp["Wn"] * g[None, :])
        wrs.append(lp["Wr"] * g[None, :])
        cs.append((lp["bn"] * g + lp["beta"])[None, :])
    wns = jnp.stack(wns)
    wrs = jnp.stack(wrs)
    cs = jnp.stack(cs)

    deg = _make_deg()(dst3d, zdeg, ones8)
    if isinstance(deg, (list, tuple)):
        deg = deg[0]

    def step(feats, ws):
        wn2, wr2, c2 = ws
        acc = _make_agg()(feats, src3, dst3, zrow)
        if isinstance(acc, (list, tuple)):
            acc = acc[0]
        return _dense(acc, deg, feats, wn2, wr2, c2), None

    feats, _ = lax.scan(step, xs, (wns, wrs, cs))
    hd = params["head"]
    return _final(feats, batch2,
                  hd["W1"], hd["b1"][None], hd["W2"], hd["b2"][None])
